# baseline (speedup 1.0000x reference)
import functools

import jax
import jax.numpy as jnp
from jax import lax
from jax.experimental import pallas as pl
from jax.experimental.pallas import tpu as pltpu
from jax.experimental.pallas import tpu_sc as plsc

BATCH = 4096
IN_F = 2048
OUT_F = 16384
NUM_CORES = 2
NUM_SUBCORES = 16
NUM_WORKERS = NUM_CORES * NUM_SUBCORES
ROWS_PER_WORKER = BATCH // NUM_WORKERS
CHUNK = 4
NUM_CHUNKS = ROWS_PER_WORKER // CHUNK

_mesh = plsc.VectorSubcoreMesh(core_axis_name="c", subcore_axis_name="s")


@functools.partial(
    pl.kernel,
    mesh=_mesh,
    out_type=jax.ShapeDtypeStruct((BATCH, OUT_F), jnp.float32),
    scratch_types=[
        pltpu.VMEM((CHUNK, IN_F), jnp.float32),
        pltpu.VMEM((CHUNK, OUT_F), jnp.float32),
    ],
)
def _rat_sc(x_hbm, out_hbm, in_v, out_v):
    wid = lax.axis_index("s") * NUM_CORES + lax.axis_index("c")
    base = wid * ROWS_PER_WORKER

    def chunk_body(g, _):
        row0 = base + g * CHUNK
        pltpu.sync_copy(x_hbm.at[pl.ds(row0, CHUNK)], in_v)
        for r in range(CHUNK):
            def f_body(f, _):
                left = in_v[r, pl.ds(f * 32, 16)]
                right = in_v[r, pl.ds(f * 32 + 16, 16)]
                for i in range(16):
                    idx = jnp.full((16, 1), i, jnp.int32)
                    li = lax.gather(
                        left, idx,
                        dimension_numbers=lax.GatherDimensionNumbers(
                            offset_dims=(), collapsed_slice_dims=(0,),
                            start_index_map=(0,)),
                        slice_sizes=(1,),
                        mode=lax.GatherScatterMode.PROMISE_IN_BOUNDS)
                    out_v[r, pl.ds(f * 256 + i * 16, 16)] = li + right
                return 0
            lax.fori_loop(0, 64, f_body, 0)
        pltpu.sync_copy(out_v, out_hbm.at[pl.ds(row0, CHUNK)])
        return 0

    lax.fori_loop(0, NUM_CHUNKS, chunk_body, 0)


def kernel(x):
    xf = x.reshape(BATCH, IN_F)
    out = _rat_sc(xf)
    return out.reshape(BATCH, 64, 256)
